# baseline (speedup 1.0000x reference)
import functools

import jax
import jax.numpy as jnp
from jax.experimental import pallas as pl

NUM_HEADS = 8
NUM_EMBEDDINGS = 512
EMBEDDING_DIM = 256
HEAD_DIM = EMBEDDING_DIM // NUM_HEADS


def _pq_kernel(z_ref, cb_ref, cbt_ref, zq_ref, idx_ref, dist_ref):
    z_blk = z_ref[...]
    zhs = [z_blk[:, h * HEAD_DIM:(h + 1) * HEAD_DIM] for h in range(NUM_HEADS)]
    dots = [jnp.dot(zhs[h], cbt_ref[h], preferred_element_type=jnp.float32)
            for h in range(NUM_HEADS)]
    dists = []
    for h in range(NUM_HEADS):
        zh = zhs[h]
        ch = cb_ref[h]
        z_sq = jnp.sum(zh * zh, axis=-1, keepdims=True)
        c_sq = jnp.sum(ch * ch, axis=-1)[None, :]
        dist = z_sq + c_sq - 2.0 * dots[h]
        dist_ref[:, h * NUM_EMBEDDINGS:(h + 1) * NUM_EMBEDDINGS] = dist
        dists.append(dist)
    idxs = [jnp.argmin(dists[h], axis=-1).astype(jnp.int32) for h in range(NUM_HEADS)]
    idx_ref[...] = jnp.concatenate([i[:, None] for i in idxs], axis=1)
    zq_cols = []
    for h in range(NUM_HEADS):
        onehot = (jax.lax.broadcasted_iota(jnp.int32, dists[h].shape, 1)
                  == idxs[h][:, None]).astype(jnp.float32)
        zq_h = jnp.dot(onehot, cb_ref[h], preferred_element_type=jnp.float32)
        zq_cols.append(zhs[h] + (zq_h - zhs[h]))
    zq_ref[...] = jnp.concatenate(zq_cols, axis=1)


@functools.partial(jax.jit, static_argnames=("block_b",))
def _pq(z, codebooks, block_b=256):
    bsz, dim = z.shape
    cbt = jnp.transpose(codebooks, (0, 2, 1))
    grid = (bsz // block_b,)
    zq, idx, dist = pl.pallas_call(
        _pq_kernel,
        grid=grid,
        in_specs=[
            pl.BlockSpec((block_b, dim), lambda i: (i, 0)),
            pl.BlockSpec((NUM_HEADS, NUM_EMBEDDINGS, HEAD_DIM), lambda i: (0, 0, 0)),
            pl.BlockSpec((NUM_HEADS, HEAD_DIM, NUM_EMBEDDINGS), lambda i: (0, 0, 0)),
        ],
        out_specs=[
            pl.BlockSpec((block_b, dim), lambda i: (i, 0)),
            pl.BlockSpec((block_b, NUM_HEADS), lambda i: (i, 0)),
            pl.BlockSpec((block_b, NUM_HEADS * NUM_EMBEDDINGS), lambda i: (i, 0)),
        ],
        out_shape=[
            jax.ShapeDtypeStruct((bsz, dim), jnp.float32),
            jax.ShapeDtypeStruct((bsz, NUM_HEADS), jnp.int32),
            jax.ShapeDtypeStruct((bsz, NUM_HEADS * NUM_EMBEDDINGS), jnp.float32),
        ],
    )(z, codebooks, cbt)
    return zq, idx, dist.reshape(bsz, NUM_HEADS, NUM_EMBEDDINGS)


def kernel(z, codebooks):
    return _pq(z, codebooks)


# speedup vs baseline: 1.2677x; 1.2677x over previous
import functools

import jax
import jax.numpy as jnp
from jax.experimental import pallas as pl

NUM_HEADS = 8
NUM_EMBEDDINGS = 512
EMBEDDING_DIM = 256
HEAD_DIM = EMBEDDING_DIM // NUM_HEADS


def _pq_kernel(z_ref, cb_ref, cbt_ref, zq_ref, idx_ref, dist_ref):
    z_blk = z_ref[...]
    zhs = [z_blk[:, h * HEAD_DIM:(h + 1) * HEAD_DIM] for h in range(NUM_HEADS)]
    for h in range(NUM_HEADS):
        zh = zhs[h]
        ch = cb_ref[h]
        dot = jnp.dot(zh, cbt_ref[h], preferred_element_type=jnp.float32)
        z_sq = jnp.sum(zh * zh, axis=-1, keepdims=True)
        c_sq = jnp.sum(ch * ch, axis=-1)[None, :]
        cols = slice(h * NUM_EMBEDDINGS, (h + 1) * NUM_EMBEDDINGS)
        dist_ref[:, cols] = z_sq + c_sq - 2.0 * dot
    idxs = []
    for h in range(NUM_HEADS):
        cols = slice(h * NUM_EMBEDDINGS, (h + 1) * NUM_EMBEDDINGS)
        m = jnp.min(dist_ref[:, cols], axis=-1, keepdims=True)
        d = dist_ref[:, cols]
        iota = jax.lax.broadcasted_iota(jnp.int32, d.shape, 1)
        idxs.append(jnp.min(jnp.where(d == m, iota, NUM_EMBEDDINGS),
                            axis=-1).astype(jnp.int32))
    idx_ref[...] = jnp.concatenate([i[:, None] for i in idxs], axis=1)
    zq_cols = []
    for h in range(NUM_HEADS):
        onehot = (jax.lax.broadcasted_iota(
            jnp.int32, (z_blk.shape[0], NUM_EMBEDDINGS), 1)
            == idxs[h][:, None]).astype(jnp.float32)
        zq_h = jnp.dot(onehot, cb_ref[h], preferred_element_type=jnp.float32)
        zq_cols.append(zhs[h] + (zq_h - zhs[h]))
    zq_ref[...] = jnp.concatenate(zq_cols, axis=1)


@functools.partial(jax.jit, static_argnames=("block_b",))
def _pq(z, codebooks, block_b=512):
    bsz, dim = z.shape
    cbt = jnp.transpose(codebooks, (0, 2, 1))
    grid = (bsz // block_b,)
    zq, idx, dist = pl.pallas_call(
        _pq_kernel,
        grid=grid,
        in_specs=[
            pl.BlockSpec((block_b, dim), lambda i: (i, 0)),
            pl.BlockSpec((NUM_HEADS, NUM_EMBEDDINGS, HEAD_DIM), lambda i: (0, 0, 0)),
            pl.BlockSpec((NUM_HEADS, HEAD_DIM, NUM_EMBEDDINGS), lambda i: (0, 0, 0)),
        ],
        out_specs=[
            pl.BlockSpec((block_b, dim), lambda i: (i, 0)),
            pl.BlockSpec((block_b, NUM_HEADS), lambda i: (i, 0)),
            pl.BlockSpec((block_b, NUM_HEADS * NUM_EMBEDDINGS), lambda i: (i, 0)),
        ],
        out_shape=[
            jax.ShapeDtypeStruct((bsz, dim), jnp.float32),
            jax.ShapeDtypeStruct((bsz, NUM_HEADS), jnp.int32),
            jax.ShapeDtypeStruct((bsz, NUM_HEADS * NUM_EMBEDDINGS), jnp.float32),
        ],
    )(z, codebooks, cbt)
    return zq, idx, dist.reshape(bsz, NUM_HEADS, NUM_EMBEDDINGS)


def kernel(z, codebooks):
    return _pq(z, codebooks)


# speedup vs baseline: 1.3499x; 1.0648x over previous
import functools

import jax
import jax.numpy as jnp
from jax.experimental import pallas as pl

NUM_HEADS = 8
NUM_EMBEDDINGS = 512
EMBEDDING_DIM = 256
HEAD_DIM = EMBEDDING_DIM // NUM_HEADS
GROUPS = 2
HEADS_PER_GROUP = NUM_HEADS // GROUPS
GROUP_DIM = HEADS_PER_GROUP * HEAD_DIM
GROUP_EMB = HEADS_PER_GROUP * NUM_EMBEDDINGS


def _pq_kernel(z_ref, cbtg_ref, mask_ref, csq_ref, cb_ref,
               zq_ref, idxp_ref, dist_ref):
    p = pl.program_id(1)
    zg = z_ref[:, pl.ds(p * GROUP_DIM, GROUP_DIM)]
    dotg = jnp.dot(zg, cbtg_ref[p], preferred_element_type=jnp.float32)
    zsqb = jnp.dot(zg * zg, mask_ref[...],
                   preferred_element_type=jnp.float32)
    dist = (zsqb + csq_ref[p][None, :]) + dotg
    dist_ref[...] = dist
    idx_cols = []
    zq_parts = []
    for j in range(HEADS_PER_GROUP):
        cols = slice(j * NUM_EMBEDDINGS, (j + 1) * NUM_EMBEDDINGS)
        m = jnp.min(dist_ref[:, cols], axis=-1, keepdims=True)
        d = dist_ref[:, cols]
        iota = jax.lax.broadcasted_iota(jnp.int32, d.shape, 1)
        idx = jnp.min(jnp.where(d == m, iota, NUM_EMBEDDINGS), axis=-1)
        idx_cols.append(idx[:, None].astype(jnp.int32))
        onehot = (iota == idx[:, None]).astype(jnp.float32)
        zq_j = jnp.dot(onehot, cb_ref[p * HEADS_PER_GROUP + j],
                       preferred_element_type=jnp.float32)
        zh = zg[:, j * HEAD_DIM:(j + 1) * HEAD_DIM]
        zq_parts.append(zh + (zq_j - zh))
    idxp_ref[0] = jnp.concatenate(idx_cols, axis=1)
    zq_ref[:, pl.ds(p * GROUP_DIM, GROUP_DIM)] = jnp.concatenate(zq_parts,
                                                                 axis=1)


@functools.partial(jax.jit, static_argnames=("block_b",))
def _pq(z, codebooks, block_b=512):
    bsz, dim = z.shape
    cbt = jnp.transpose(codebooks, (0, 2, 1))
    cbtg = jnp.zeros((GROUPS, HEADS_PER_GROUP, HEAD_DIM,
                      HEADS_PER_GROUP, NUM_EMBEDDINGS), jnp.float32)
    cbtr = cbt.reshape(GROUPS, HEADS_PER_GROUP, HEAD_DIM, NUM_EMBEDDINGS)
    for j in range(HEADS_PER_GROUP):
        cbtg = cbtg.at[:, j, :, j, :].set(-2.0 * cbtr[:, j])
    cbtg = cbtg.reshape(GROUPS, GROUP_DIM, GROUP_EMB)
    mask = (jax.lax.broadcasted_iota(jnp.int32, (GROUP_DIM, GROUP_EMB), 0)
            // HEAD_DIM ==
            jax.lax.broadcasted_iota(jnp.int32, (GROUP_DIM, GROUP_EMB), 1)
            // NUM_EMBEDDINGS).astype(jnp.float32)
    csq = jnp.sum(codebooks ** 2, axis=-1).reshape(GROUPS, GROUP_EMB)
    grid = (bsz // block_b, GROUPS)
    zq, idxp, dist = pl.pallas_call(
        _pq_kernel,
        grid=grid,
        in_specs=[
            pl.BlockSpec((block_b, dim), lambda i, p: (i, 0)),
            pl.BlockSpec((GROUPS, GROUP_DIM, GROUP_EMB),
                         lambda i, p: (0, 0, 0)),
            pl.BlockSpec((GROUP_DIM, GROUP_EMB), lambda i, p: (0, 0)),
            pl.BlockSpec((GROUPS, GROUP_EMB), lambda i, p: (0, 0)),
            pl.BlockSpec((NUM_HEADS, NUM_EMBEDDINGS, HEAD_DIM),
                         lambda i, p: (0, 0, 0)),
        ],
        out_specs=[
            pl.BlockSpec((block_b, dim), lambda i, p: (i, 0)),
            pl.BlockSpec((1, block_b, HEADS_PER_GROUP), lambda i, p: (p, i, 0)),
            pl.BlockSpec((block_b, GROUP_EMB), lambda i, p: (i, p)),
        ],
        out_shape=[
            jax.ShapeDtypeStruct((bsz, dim), jnp.float32),
            jax.ShapeDtypeStruct((GROUPS, bsz, HEADS_PER_GROUP), jnp.int32),
            jax.ShapeDtypeStruct((bsz, NUM_HEADS * NUM_EMBEDDINGS), jnp.float32),
        ],
    )(z, cbtg, mask, csq, codebooks)
    idx = jnp.transpose(idxp, (1, 0, 2)).reshape(bsz, NUM_HEADS)
    return zq, idx, dist.reshape(bsz, NUM_HEADS, NUM_EMBEDDINGS)


def kernel(z, codebooks):
    return _pq(z, codebooks)


# speedup vs baseline: 1.4797x; 1.0962x over previous
import functools

import jax
import jax.numpy as jnp
from jax.experimental import pallas as pl

NUM_HEADS = 8
NUM_EMBEDDINGS = 512
EMBEDDING_DIM = 256
HEAD_DIM = EMBEDDING_DIM // NUM_HEADS
GROUPS = 2
HEADS_PER_GROUP = NUM_HEADS // GROUPS
GROUP_DIM = HEADS_PER_GROUP * HEAD_DIM
GROUP_EMB = HEADS_PER_GROUP * NUM_EMBEDDINGS


def _pq_kernel(z_ref, wg_ref, cb_ref, zq_ref, idxp_ref, dist_ref):
    p = pl.program_id(1)
    zg = z_ref[:, pl.ds(p * GROUP_DIM, GROUP_DIM)]
    ones = jnp.ones((zg.shape[0], 8), jnp.float32)
    zaug = jnp.concatenate([zg, zg * zg, ones], axis=1)
    dist = jnp.dot(zaug, wg_ref[p], preferred_element_type=jnp.float32)
    dist_ref[...] = dist
    idx_cols = []
    zq_parts = []
    for j in range(HEADS_PER_GROUP):
        cols = slice(j * NUM_EMBEDDINGS, (j + 1) * NUM_EMBEDDINGS)
        m = jnp.min(dist_ref[:, cols], axis=-1, keepdims=True)
        d = dist_ref[:, cols]
        iota = jax.lax.broadcasted_iota(jnp.int32, d.shape, 1)
        idx = jnp.min(jnp.where(d == m, iota, NUM_EMBEDDINGS), axis=-1)
        idx_cols.append(idx[:, None].astype(jnp.int32))
        onehot = (iota == idx[:, None]).astype(jnp.float32)
        zq_j = jnp.dot(onehot, cb_ref[p * HEADS_PER_GROUP + j],
                       preferred_element_type=jnp.float32)
        zh = zg[:, j * HEAD_DIM:(j + 1) * HEAD_DIM]
        zq_parts.append(zh + (zq_j - zh))
    idxp_ref[0] = jnp.concatenate(idx_cols, axis=1)
    zq_ref[:, pl.ds(p * GROUP_DIM, GROUP_DIM)] = jnp.concatenate(zq_parts,
                                                                 axis=1)


@functools.partial(jax.jit, static_argnames=("block_b",))
def _pq(z, codebooks, block_b=512):
    bsz, dim = z.shape
    cbt = jnp.transpose(codebooks, (0, 2, 1))
    cbtg = jnp.zeros((GROUPS, HEADS_PER_GROUP, HEAD_DIM,
                      HEADS_PER_GROUP, NUM_EMBEDDINGS), jnp.float32)
    cbtr = cbt.reshape(GROUPS, HEADS_PER_GROUP, HEAD_DIM, NUM_EMBEDDINGS)
    for j in range(HEADS_PER_GROUP):
        cbtg = cbtg.at[:, j, :, j, :].set(-2.0 * cbtr[:, j])
    cbtg = cbtg.reshape(GROUPS, GROUP_DIM, GROUP_EMB)
    mask = (jax.lax.broadcasted_iota(jnp.int32, (GROUP_DIM, GROUP_EMB), 0)
            // HEAD_DIM ==
            jax.lax.broadcasted_iota(jnp.int32, (GROUP_DIM, GROUP_EMB), 1)
            // NUM_EMBEDDINGS).astype(jnp.float32)
    mask = jnp.broadcast_to(mask[None], (GROUPS, GROUP_DIM, GROUP_EMB))
    csq = jnp.sum(codebooks ** 2, axis=-1).reshape(GROUPS, 1, GROUP_EMB)
    zpad = jnp.zeros((GROUPS, 7, GROUP_EMB), jnp.float32)
    wg = jnp.concatenate([cbtg, mask, csq, zpad], axis=1)
    grid = (bsz // block_b, GROUPS)
    zq, idxp, dist = pl.pallas_call(
        _pq_kernel,
        grid=grid,
        in_specs=[
            pl.BlockSpec((block_b, dim), lambda i, p: (i, 0)),
            pl.BlockSpec((GROUPS, 264, GROUP_EMB), lambda i, p: (0, 0, 0)),
            pl.BlockSpec((NUM_HEADS, NUM_EMBEDDINGS, HEAD_DIM),
                         lambda i, p: (0, 0, 0)),
        ],
        out_specs=[
            pl.BlockSpec((block_b, dim), lambda i, p: (i, 0)),
            pl.BlockSpec((1, block_b, HEADS_PER_GROUP), lambda i, p: (p, i, 0)),
            pl.BlockSpec((block_b, GROUP_EMB), lambda i, p: (i, p)),
        ],
        out_shape=[
            jax.ShapeDtypeStruct((bsz, dim), jnp.float32),
            jax.ShapeDtypeStruct((GROUPS, bsz, HEADS_PER_GROUP), jnp.int32),
            jax.ShapeDtypeStruct((bsz, NUM_HEADS * NUM_EMBEDDINGS), jnp.float32),
        ],
    )(z, wg, codebooks)
    idx = jnp.transpose(idxp, (1, 0, 2)).reshape(bsz, NUM_HEADS)
    return zq, idx, dist.reshape(bsz, NUM_HEADS, NUM_EMBEDDINGS)


def kernel(z, codebooks):
    return _pq(z, codebooks)
